# Initial kernel scaffold; baseline (speedup 1.0000x reference)
import jax
import jax.numpy as jnp
from jax.experimental import pallas as pl


def kernel(x, edge_index, W, b, lin_W, lin_b):
    raise NotImplementedError("write your pallas kernel here")


# speedup vs baseline: 40.3651x; 40.3651x over previous
import functools

import jax
import jax.numpy as jnp
from jax import lax
from jax.experimental import pallas as pl
from jax.experimental.pallas import tpu as pltpu
from jax.experimental.pallas import tpu_sc as plsc

N = 10000
E = 320000
F_IN = 128
F_OUT = 20
FP = 24
NUM_CLASSES = 10

NC = 2
NS = 16
NW = NC * NS
PW = E // NW
W0 = 80
NWIN = PW // W0

_mesh = plsc.VectorSubcoreMesh(core_axis_name="c", subcore_axis_name="s")
_CP = pltpu.CompilerParams(use_tc_tiling_on_sc=False)


@functools.partial(
    pl.kernel,
    mesh=_mesh,
    out_type=jax.ShapeDtypeStruct((NC, N), jnp.float32),
    scratch_types=[
        pltpu.VMEM((NWIN, W0), jnp.int32),
        pltpu.VMEM((W0,), jnp.float32),
        pltpu.VMEM_SHARED((N,), jnp.float32),
        pltpu.SemaphoreType.DMA,
    ],
    compiler_params=_CP,
)
def _deg_kernel(dst_hbm, zeros_hbm, out_hbm, idx_v, ones_v, deg_sh, sem):
    c = lax.axis_index("c")
    s = lax.axis_index("s")
    wid = s * NC + c

    @pl.when(s == 0)
    def _():
        pltpu.sync_copy(zeros_hbm, deg_sh)

    for i in range(W0 // 16):
        ones_v[pl.ds(i * 16, 16)] = jnp.ones((16,), jnp.float32)
    pltpu.sync_copy(dst_hbm.at[wid], idx_v)
    plsc.subcore_barrier()

    def w_body(w, carry):
        pltpu.sync_copy(ones_v, deg_sh.at[idx_v.at[w]], add=True)
        return carry

    lax.fori_loop(0, NWIN, w_body, 0)
    plsc.subcore_barrier()

    @pl.when(s == 0)
    def _():
        pltpu.sync_copy(deg_sh, out_hbm.at[c])


@functools.partial(
    pl.kernel,
    mesh=_mesh,
    out_type=jax.ShapeDtypeStruct((NC, N, FP), jnp.float32),
    scratch_types=[
        pltpu.VMEM((NWIN, W0), jnp.int32),
        pltpu.VMEM((NWIN, W0), jnp.int32),
        pltpu.VMEM((W0, FP), jnp.float32),
        pltpu.VMEM_SHARED((N, FP), jnp.float32),
        pltpu.SemaphoreType.DMA,
    ],
    compiler_params=_CP,
)
def _scatter_kernel(src_hbm, dst_hbm, g_hbm, zeros_hbm, out_hbm,
                    sidx_v, didx_v, rows_v, acc_sh, sem):
    c = lax.axis_index("c")
    s = lax.axis_index("s")
    wid = s * NC + c

    @pl.when(s == 0)
    def _():
        pltpu.sync_copy(zeros_hbm, acc_sh)

    pltpu.sync_copy(src_hbm.at[wid], sidx_v)
    pltpu.sync_copy(dst_hbm.at[wid], didx_v)
    plsc.subcore_barrier()

    def w_body(w, carry):
        pltpu.async_copy(g_hbm.at[sidx_v.at[w]], rows_v, sem).wait()
        pltpu.sync_copy(rows_v, acc_sh.at[didx_v.at[w]], add=True)
        return carry

    lax.fori_loop(0, NWIN, w_body, 0)
    plsc.subcore_barrier()

    @pl.when(s == 0)
    def _():
        pltpu.sync_copy(acc_sh, out_hbm.at[c])


def _prep_body(x_ref, w_ref, degp_ref, g_ref):
    deg = degp_ref[0, :] + degp_ref[1, :] + 1.0
    dis = lax.rsqrt(deg)[:, None]
    h = jnp.dot(x_ref[...], w_ref[...], preferred_element_type=jnp.float32)
    g_ref[...] = h * dis


def _fin_body(ap_ref, g_ref, degp_ref, b_ref, lw_ref, lb_ref, o_ref):
    deg = degp_ref[0, :] + degp_ref[1, :] + 1.0
    dis = lax.rsqrt(deg)[:, None]
    a = (ap_ref[0] + ap_ref[1] + g_ref[...])[:, :F_OUT]
    out1 = a * dis + b_ref[...]
    nrm = jnp.sqrt(jnp.sum(out1 * out1, axis=1, keepdims=True))
    out1 = out1 / jnp.maximum(nrm, 1e-12)
    e = jnp.maximum(out1, 0.0)
    pmax = jnp.max(e, axis=0, keepdims=True)
    pmean = jnp.sum(e, axis=0, keepdims=True) * (1.0 / N)
    feat = jnp.concatenate([pmax, pmean], axis=1)
    o_ref[...] = jnp.dot(feat, lw_ref[...],
                         preferred_element_type=jnp.float32) + lb_ref[...]


def kernel(x, edge_index, W, b, lin_W, lin_b):
    src = edge_index[0].reshape(NW, NWIN, W0)
    dst = edge_index[1].reshape(NW, NWIN, W0)
    zeros_n = jnp.zeros((N,), jnp.float32)
    zeros_nf = jnp.zeros((N, FP), jnp.float32)
    w_pad = jnp.pad(W, ((0, 0), (0, FP - F_OUT)))

    degp = _deg_kernel(dst, zeros_n)

    g = pl.pallas_call(
        _prep_body,
        out_shape=jax.ShapeDtypeStruct((N, FP), jnp.float32),
    )(x, w_pad, degp)

    ap = _scatter_kernel(src, dst, g, zeros_nf)

    out = pl.pallas_call(
        _fin_body,
        out_shape=jax.ShapeDtypeStruct((1, NUM_CLASSES), jnp.float32),
    )(ap, g, degp, b.reshape(1, F_OUT), lin_W, lin_b.reshape(1, NUM_CLASSES))
    return out


# speedup vs baseline: 53.7007x; 1.3304x over previous
import functools

import jax
import jax.numpy as jnp
from jax import lax
from jax.experimental import pallas as pl
from jax.experimental.pallas import tpu as pltpu
from jax.experimental.pallas import tpu_sc as plsc

N = 10000
E = 320000
F_IN = 128
F_OUT = 20
FP = 24
NUM_CLASSES = 10

NC = 2
NS = 16
NW = NC * NS
PW = E // NW
W0 = 80
NWIN = PW // W0

_mesh = plsc.VectorSubcoreMesh(core_axis_name="c", subcore_axis_name="s")
_CP = pltpu.CompilerParams(use_tc_tiling_on_sc=False)


@functools.partial(
    pl.kernel,
    mesh=_mesh,
    out_type=jax.ShapeDtypeStruct((NC, N), jnp.float32),
    scratch_types=[
        pltpu.VMEM((NWIN, W0), jnp.int32),
        pltpu.VMEM((W0,), jnp.float32),
        pltpu.VMEM_SHARED((N,), jnp.float32),
        pltpu.SemaphoreType.DMA,
    ],
    compiler_params=_CP,
)
def _deg_kernel(dst_hbm, zeros_hbm, out_hbm, idx_v, ones_v, deg_sh, sem):
    c = lax.axis_index("c")
    s = lax.axis_index("s")
    wid = s * NC + c

    @pl.when(s == 0)
    def _():
        pltpu.sync_copy(zeros_hbm, deg_sh)

    for i in range(W0 // 16):
        ones_v[pl.ds(i * 16, 16)] = jnp.ones((16,), jnp.float32)
    pltpu.sync_copy(dst_hbm.at[wid], idx_v)
    plsc.subcore_barrier()

    def w_body(w, carry):
        pltpu.sync_copy(ones_v, deg_sh.at[idx_v.at[w]], add=True)
        return carry

    lax.fori_loop(0, NWIN, w_body, 0)
    plsc.subcore_barrier()

    @pl.when(s == 0)
    def _():
        pltpu.sync_copy(deg_sh, out_hbm.at[c])


@functools.partial(
    pl.kernel,
    mesh=_mesh,
    out_type=jax.ShapeDtypeStruct((NC, N, FP), jnp.float32),
    scratch_types=[
        pltpu.VMEM((NWIN, W0), jnp.int32),
        pltpu.VMEM((NWIN, W0), jnp.int32),
        pltpu.VMEM((2, W0, FP), jnp.float32),
        pltpu.VMEM_SHARED((N, FP), jnp.float32),
        pltpu.SemaphoreType.DMA,
    ],
    compiler_params=_CP,
)
def _scatter_kernel(src_hbm, dst_hbm, g_hbm, zeros_hbm, out_hbm,
                    sidx_v, didx_v, rows_v, acc_sh, sem):
    c = lax.axis_index("c")
    s = lax.axis_index("s")
    wid = s * NC + c

    @pl.when((s == 0) & (c == 0))
    def _():
        pltpu.sync_copy(g_hbm, acc_sh)

    @pl.when((s == 0) & (c == 1))
    def _():
        pltpu.sync_copy(zeros_hbm, acc_sh)

    pltpu.sync_copy(src_hbm.at[wid], sidx_v)
    pltpu.sync_copy(dst_hbm.at[wid], didx_v)
    plsc.subcore_barrier()

    pltpu.async_copy(g_hbm.at[sidx_v.at[0]], rows_v.at[0], sem)

    def w_body(w, carry):
        nxt = w + 1

        @pl.when(nxt < NWIN)
        def _():
            pltpu.async_copy(g_hbm.at[sidx_v.at[nxt]],
                             rows_v.at[nxt % 2], sem)

        pltpu.make_async_copy(g_hbm.at[sidx_v.at[w]],
                              rows_v.at[w % 2], sem).wait()
        pltpu.sync_copy(rows_v.at[w % 2], acc_sh.at[didx_v.at[w]], add=True)
        return carry

    lax.fori_loop(0, NWIN, w_body, 0)
    plsc.subcore_barrier()

    @pl.when(s == 0)
    def _():
        pltpu.sync_copy(acc_sh, out_hbm.at[c])


def _prep_body(x_ref, w_ref, degp_ref, g_ref):
    deg = degp_ref[0, :] + degp_ref[1, :] + 1.0
    dis = lax.rsqrt(deg)[:, None]
    h = jnp.dot(x_ref[...], w_ref[...], preferred_element_type=jnp.float32)
    g_ref[...] = h * dis


def _fin_body(ap_ref, degp_ref, b_ref, lw_ref, lb_ref, o_ref):
    deg = degp_ref[0, :] + degp_ref[1, :] + 1.0
    dis = lax.rsqrt(deg)[:, None]
    a = (ap_ref[0] + ap_ref[1])[:, :F_OUT]
    out1 = a * dis + b_ref[...]
    nrm = jnp.sqrt(jnp.sum(out1 * out1, axis=1, keepdims=True))
    out1 = out1 / jnp.maximum(nrm, 1e-12)
    e = jnp.maximum(out1, 0.0)
    pmax = jnp.max(e, axis=0, keepdims=True)
    pmean = jnp.sum(e, axis=0, keepdims=True) * (1.0 / N)
    feat = jnp.concatenate([pmax, pmean], axis=1)
    o_ref[...] = jnp.dot(feat, lw_ref[...],
                         preferred_element_type=jnp.float32) + lb_ref[...]


def kernel(x, edge_index, W, b, lin_W, lin_b):
    src = edge_index[0].reshape(NW, NWIN, W0)
    dst = edge_index[1].reshape(NW, NWIN, W0)
    zeros_n = jnp.zeros((N,), jnp.float32)
    zeros_nf = jnp.zeros((N, FP), jnp.float32)
    w_pad = jnp.pad(W, ((0, 0), (0, FP - F_OUT)))

    degp = _deg_kernel(dst, zeros_n)

    g = pl.pallas_call(
        _prep_body,
        out_shape=jax.ShapeDtypeStruct((N, FP), jnp.float32),
    )(x, w_pad, degp)

    ap = _scatter_kernel(src, dst, g, zeros_nf)

    out = pl.pallas_call(
        _fin_body,
        out_shape=jax.ShapeDtypeStruct((1, NUM_CLASSES), jnp.float32),
    )(ap, degp, b.reshape(1, F_OUT), lin_W, lin_b.reshape(1, NUM_CLASSES))
    return out


# speedup vs baseline: 59.7438x; 1.1125x over previous
import functools

import jax
import jax.numpy as jnp
import numpy as np
from jax import lax
from jax.experimental import pallas as pl
from jax.experimental.pallas import tpu as pltpu
from jax.experimental.pallas import tpu_sc as plsc

N = 10000
E = 320000
F_IN = 128
F_OUT = 20
FP = 24
NUM_CLASSES = 10

NC = 2
NS = 16
NW = NC * NS
W0 = 128
NWIN = 79
EP = NW * NWIN * W0
NT = 128
NA = N + NT

D_RING = 4
P_PRE = 2

_mesh = plsc.VectorSubcoreMesh(core_axis_name="c", subcore_axis_name="s")
_CP = pltpu.CompilerParams(use_tc_tiling_on_sc=False)


@functools.partial(
    pl.kernel,
    mesh=_mesh,
    out_type=jax.ShapeDtypeStruct((NC, NA), jnp.float32),
    scratch_types=[
        pltpu.VMEM((NWIN, W0), jnp.int32),
        pltpu.VMEM((W0,), jnp.float32),
        pltpu.VMEM_SHARED((NA,), jnp.float32),
        pltpu.SemaphoreType.DMA,
    ],
    compiler_params=_CP,
)
def _deg_kernel(dst_hbm, zeros_hbm, out_hbm, idx_v, ones_v, deg_sh, sem):
    c = lax.axis_index("c")
    s = lax.axis_index("s")
    wid = s * NC + c

    @pl.when(s == 0)
    def _():
        pltpu.sync_copy(zeros_hbm, deg_sh)

    for i in range(W0 // 16):
        ones_v[pl.ds(i * 16, 16)] = jnp.ones((16,), jnp.float32)
    pltpu.sync_copy(dst_hbm.at[wid], idx_v)
    plsc.subcore_barrier()

    def fire(w, carry):
        pltpu.async_copy(ones_v, deg_sh.at[idx_v.at[w]], sem, add=True)
        return carry

    lax.fori_loop(0, NWIN, fire, 0)

    def drain(w, carry):
        pltpu.make_async_copy(ones_v, deg_sh.at[idx_v.at[0]], sem).wait()
        return carry

    lax.fori_loop(0, NWIN, drain, 0)
    plsc.subcore_barrier()

    @pl.when(s == 0)
    def _():
        pltpu.sync_copy(deg_sh, out_hbm.at[c])


@functools.partial(
    pl.kernel,
    mesh=_mesh,
    out_type=jax.ShapeDtypeStruct((NC, NA, FP), jnp.float32),
    scratch_types=[
        pltpu.VMEM((NWIN, W0), jnp.int32),
        pltpu.VMEM((NWIN, W0), jnp.int32),
        pltpu.VMEM((D_RING, W0, FP), jnp.float32),
        pltpu.VMEM_SHARED((NA, FP), jnp.float32),
        pltpu.SemaphoreType.DMA,
        pltpu.SemaphoreType.DMA,
    ],
    compiler_params=_CP,
)
def _scatter_kernel(src_hbm, dst_hbm, g_hbm, zeros_hbm, out_hbm,
                    sidx_v, didx_v, rows_v, acc_sh, sem_g, sem_s):
    c = lax.axis_index("c")
    s = lax.axis_index("s")
    wid = s * NC + c

    @pl.when((s == 0) & (c == 0))
    def _():
        pltpu.sync_copy(g_hbm, acc_sh)

    @pl.when((s == 0) & (c == 1))
    def _():
        pltpu.sync_copy(zeros_hbm, acc_sh)

    pltpu.sync_copy(src_hbm.at[wid], sidx_v)
    pltpu.sync_copy(dst_hbm.at[wid], didx_v)
    plsc.subcore_barrier()

    for i in range(P_PRE):
        pltpu.async_copy(g_hbm.at[sidx_v.at[i]], rows_v.at[i], sem_g)

    def w_body(w, carry):
        @pl.when(w >= P_PRE)
        def _():
            pltpu.make_async_copy(rows_v.at[0], acc_sh.at[didx_v.at[0]],
                                  sem_s).wait()

        pltpu.make_async_copy(g_hbm.at[sidx_v.at[w]],
                              rows_v.at[w % D_RING], sem_g).wait()
        pltpu.async_copy(rows_v.at[w % D_RING],
                         acc_sh.at[didx_v.at[w]], sem_s, add=True)
        nxt = w + P_PRE

        @pl.when(nxt < NWIN)
        def _():
            pltpu.async_copy(g_hbm.at[sidx_v.at[nxt]],
                             rows_v.at[nxt % D_RING], sem_g)

        return carry

    lax.fori_loop(0, NWIN, w_body, 0)
    for _ in range(P_PRE):
        pltpu.make_async_copy(rows_v.at[0], acc_sh.at[didx_v.at[0]],
                              sem_s).wait()
    plsc.subcore_barrier()

    @pl.when(s == 0)
    def _():
        pltpu.sync_copy(acc_sh, out_hbm.at[c])


def _prep_body(x_ref, w_ref, degp_ref, g_ref):
    deg = degp_ref[0, :N] + degp_ref[1, :N] + 1.0
    dis = lax.rsqrt(deg)[:, None]
    h = jnp.dot(x_ref[...], w_ref[...], preferred_element_type=jnp.float32)
    g_ref[pl.ds(0, N), :] = h * dis
    g_ref[pl.ds(N, NT), :] = jnp.zeros((NT, FP), jnp.float32)


def _fin_body(ap_ref, degp_ref, b_ref, lw_ref, lb_ref, o_ref):
    deg = degp_ref[0, :N] + degp_ref[1, :N] + 1.0
    dis = lax.rsqrt(deg)[:, None]
    a = (ap_ref[0, :N, :F_OUT] + ap_ref[1, :N, :F_OUT])
    out1 = a * dis + b_ref[...]
    nrm = jnp.sqrt(jnp.sum(out1 * out1, axis=1, keepdims=True))
    out1 = out1 / jnp.maximum(nrm, 1e-12)
    e = jnp.maximum(out1, 0.0)
    pmax = jnp.max(e, axis=0, keepdims=True)
    pmean = jnp.sum(e, axis=0, keepdims=True) * (1.0 / N)
    feat = jnp.concatenate([pmax, pmean], axis=1)
    o_ref[...] = jnp.dot(feat, lw_ref[...],
                         preferred_element_type=jnp.float32) + lb_ref[...]


_PAD = EP - E
_pad_src = ((np.arange(_PAD, dtype=np.int64) * 79) % N).astype(np.int32)
_pad_dst = (N + np.arange(_PAD, dtype=np.int64) % NT).astype(np.int32)


def kernel(x, edge_index, W, b, lin_W, lin_b):
    src = jnp.concatenate([edge_index[0], _pad_src]).reshape(NW, NWIN, W0)
    dst = jnp.concatenate([edge_index[1], _pad_dst]).reshape(NW, NWIN, W0)
    zeros_n = jnp.zeros((NA,), jnp.float32)
    zeros_nf = jnp.zeros((NA, FP), jnp.float32)
    w_pad = jnp.pad(W, ((0, 0), (0, FP - F_OUT)))

    degp = _deg_kernel(dst, zeros_n)

    g = pl.pallas_call(
        _prep_body,
        out_shape=jax.ShapeDtypeStruct((NA, FP), jnp.float32),
    )(x, w_pad, degp)

    ap = _scatter_kernel(src, dst, g, zeros_nf)

    out = pl.pallas_call(
        _fin_body,
        out_shape=jax.ShapeDtypeStruct((1, NUM_CLASSES), jnp.float32),
    )(ap, degp, b.reshape(1, F_OUT), lin_W, lin_b.reshape(1, NUM_CLASSES))
    return out


# speedup vs baseline: 70.4265x; 1.1788x over previous
import functools

import jax
import jax.numpy as jnp
import numpy as np
from jax import lax
from jax.experimental import pallas as pl
from jax.experimental.pallas import tpu as pltpu
from jax.experimental.pallas import tpu_sc as plsc

N = 10000
E = 320000
F_IN = 128
F_OUT = 20
FP = 24
NUM_CLASSES = 10

NC = 2
NS = 16
NW = NC * NS
W0 = 128
NWIN = 79
EP = NW * NWIN * W0
NT = 128
NA = N + NT

D_RING = 6
P_PRE = 3

_mesh = plsc.VectorSubcoreMesh(core_axis_name="c", subcore_axis_name="s")
_CP = pltpu.CompilerParams(use_tc_tiling_on_sc=False)


@functools.partial(
    pl.kernel,
    mesh=_mesh,
    out_type=jax.ShapeDtypeStruct((NC, NA), jnp.float32),
    scratch_types=[
        pltpu.VMEM((NWIN, W0), jnp.int32),
        pltpu.VMEM((W0,), jnp.float32),
        pltpu.VMEM_SHARED((NA,), jnp.float32),
        pltpu.SemaphoreType.DMA,
    ],
    compiler_params=_CP,
)
def _deg_kernel(dst_hbm, zeros_hbm, out_hbm, idx_v, ones_v, deg_sh, sem):
    c = lax.axis_index("c")
    s = lax.axis_index("s")
    wid = s * NC + c

    @pl.when(s == 0)
    def _():
        pltpu.sync_copy(zeros_hbm, deg_sh)

    for i in range(W0 // 16):
        ones_v[pl.ds(i * 16, 16)] = jnp.ones((16,), jnp.float32)
    pltpu.sync_copy(dst_hbm.at[wid], idx_v)
    plsc.subcore_barrier()

    def fire(w, carry):
        pltpu.async_copy(ones_v, deg_sh.at[idx_v.at[w]], sem, add=True)
        return carry

    lax.fori_loop(0, NWIN, fire, 0)

    def drain(w, carry):
        pltpu.make_async_copy(ones_v, deg_sh.at[idx_v.at[0]], sem).wait()
        return carry

    lax.fori_loop(0, NWIN, drain, 0)
    plsc.subcore_barrier()

    @pl.when(s == 0)
    def _():
        pltpu.sync_copy(deg_sh, out_hbm.at[c])


@functools.partial(
    pl.kernel,
    mesh=_mesh,
    out_type=jax.ShapeDtypeStruct((NC, NA, FP), jnp.float32),
    scratch_types=[
        pltpu.VMEM((NWIN, W0), jnp.int32),
        pltpu.VMEM((NWIN, W0), jnp.int32),
        pltpu.VMEM((D_RING, W0, FP), jnp.float32),
        pltpu.VMEM_SHARED((NA, FP), jnp.float32),
        pltpu.SemaphoreType.DMA,
        pltpu.SemaphoreType.DMA,
    ],
    compiler_params=_CP,
)
def _scatter_kernel(src_hbm, dst_hbm, g_hbm, zeros_hbm, out_hbm,
                    sidx_v, didx_v, rows_v, acc_sh, sem_g, sem_s):
    c = lax.axis_index("c")
    s = lax.axis_index("s")
    wid = s * NC + c

    @pl.when((s == 0) & (c == 0))
    def _():
        pltpu.sync_copy(g_hbm, acc_sh)

    @pl.when((s == 0) & (c == 1))
    def _():
        pltpu.sync_copy(zeros_hbm, acc_sh)

    pltpu.sync_copy(src_hbm.at[wid], sidx_v)
    pltpu.sync_copy(dst_hbm.at[wid], didx_v)
    plsc.subcore_barrier()

    for i in range(P_PRE):
        pltpu.async_copy(g_hbm.at[sidx_v.at[i]], rows_v.at[i], sem_g)

    def w_body(w, carry):
        @pl.when(w >= P_PRE)
        def _():
            pltpu.make_async_copy(rows_v.at[0], acc_sh.at[didx_v.at[0]],
                                  sem_s).wait()

        pltpu.make_async_copy(g_hbm.at[sidx_v.at[w]],
                              rows_v.at[w % D_RING], sem_g).wait()
        pltpu.async_copy(rows_v.at[w % D_RING],
                         acc_sh.at[didx_v.at[w]], sem_s, add=True)
        nxt = w + P_PRE

        @pl.when(nxt < NWIN)
        def _():
            pltpu.async_copy(g_hbm.at[sidx_v.at[nxt]],
                             rows_v.at[nxt % D_RING], sem_g)

        return carry

    lax.fori_loop(0, NWIN, w_body, 0)
    for _ in range(P_PRE):
        pltpu.make_async_copy(rows_v.at[0], acc_sh.at[didx_v.at[0]],
                              sem_s).wait()
    plsc.subcore_barrier()

    @pl.when(s == 0)
    def _():
        pltpu.sync_copy(acc_sh, out_hbm.at[c])


def _prep_body(x_ref, w_ref, degp_ref, g_ref):
    deg = degp_ref[0, :N] + degp_ref[1, :N] + 1.0
    dis = lax.rsqrt(deg)[:, None]
    h = jnp.dot(x_ref[...], w_ref[...], preferred_element_type=jnp.float32)
    g_ref[pl.ds(0, N), :] = h * dis
    g_ref[pl.ds(N, NT), :] = jnp.zeros((NT, FP), jnp.float32)


def _fin_body(ap_ref, degp_ref, b_ref, lw_ref, lb_ref, o_ref):
    deg = degp_ref[0, :N] + degp_ref[1, :N] + 1.0
    dis = lax.rsqrt(deg)[:, None]
    a = (ap_ref[0, :N, :F_OUT] + ap_ref[1, :N, :F_OUT])
    out1 = a * dis + b_ref[...]
    nrm = jnp.sqrt(jnp.sum(out1 * out1, axis=1, keepdims=True))
    out1 = out1 / jnp.maximum(nrm, 1e-12)
    e = jnp.maximum(out1, 0.0)
    pmax = jnp.max(e, axis=0, keepdims=True)
    pmean = jnp.sum(e, axis=0, keepdims=True) * (1.0 / N)
    feat = jnp.concatenate([pmax, pmean], axis=1)
    o_ref[...] = jnp.dot(feat, lw_ref[...],
                         preferred_element_type=jnp.float32) + lb_ref[...]


_PAD = EP - E
_pad_src = ((np.arange(_PAD, dtype=np.int64) * 79) % N).astype(np.int32)
_pad_dst = (N + np.arange(_PAD, dtype=np.int64) % NT).astype(np.int32)


def kernel(x, edge_index, W, b, lin_W, lin_b):
    src = jnp.concatenate([edge_index[0], _pad_src]).reshape(NW, NWIN, W0)
    dst = jnp.concatenate([edge_index[1], _pad_dst]).reshape(NW, NWIN, W0)
    zeros_n = jnp.zeros((NA,), jnp.float32)
    zeros_nf = jnp.zeros((NA, FP), jnp.float32)
    w_pad = jnp.pad(W, ((0, 0), (0, FP - F_OUT)))

    degp = _deg_kernel(dst, zeros_n)

    g = pl.pallas_call(
        _prep_body,
        out_shape=jax.ShapeDtypeStruct((NA, FP), jnp.float32),
    )(x, w_pad, degp)

    ap = _scatter_kernel(src, dst, g, zeros_nf)

    out = pl.pallas_call(
        _fin_body,
        out_shape=jax.ShapeDtypeStruct((1, NUM_CLASSES), jnp.float32),
    )(ap, degp, b.reshape(1, F_OUT), lin_W, lin_b.reshape(1, NUM_CLASSES))
    return out


# speedup vs baseline: 77.6223x; 1.1022x over previous
import functools

import jax
import jax.numpy as jnp
import numpy as np
from jax import lax
from jax.experimental import pallas as pl
from jax.experimental.pallas import tpu as pltpu
from jax.experimental.pallas import tpu_sc as plsc

N = 10000
E = 320000
F_IN = 128
F_OUT = 20
FP = 24
NUM_CLASSES = 10

NC = 2
NS = 16
NW = NC * NS
W0 = 128
NWIN = 79
EP = NW * NWIN * W0
NT = 128
NA = N + NT

D_RING = 10
P_PRE = 5

_mesh = plsc.VectorSubcoreMesh(core_axis_name="c", subcore_axis_name="s")
_CP = pltpu.CompilerParams(use_tc_tiling_on_sc=False)


@functools.partial(
    pl.kernel,
    mesh=_mesh,
    out_type=jax.ShapeDtypeStruct((NC, NA), jnp.float32),
    scratch_types=[
        pltpu.VMEM((NWIN, W0), jnp.int32),
        pltpu.VMEM((W0,), jnp.float32),
        pltpu.VMEM_SHARED((NA,), jnp.float32),
        pltpu.SemaphoreType.DMA,
    ],
    compiler_params=_CP,
)
def _deg_kernel(dst_hbm, zeros_hbm, out_hbm, idx_v, ones_v, deg_sh, sem):
    c = lax.axis_index("c")
    s = lax.axis_index("s")
    wid = s * NC + c

    @pl.when(s == 0)
    def _():
        pltpu.sync_copy(zeros_hbm, deg_sh)

    for i in range(W0 // 16):
        ones_v[pl.ds(i * 16, 16)] = jnp.ones((16,), jnp.float32)
    pltpu.sync_copy(dst_hbm.at[wid], idx_v)
    plsc.subcore_barrier()

    def fire(w, carry):
        pltpu.async_copy(ones_v, deg_sh.at[idx_v.at[w]], sem, add=True)
        return carry

    lax.fori_loop(0, NWIN, fire, 0)

    def drain(w, carry):
        pltpu.make_async_copy(ones_v, deg_sh.at[idx_v.at[0]], sem).wait()
        return carry

    lax.fori_loop(0, NWIN, drain, 0)
    plsc.subcore_barrier()

    @pl.when(s == 0)
    def _():
        pltpu.sync_copy(deg_sh, out_hbm.at[c])


@functools.partial(
    pl.kernel,
    mesh=_mesh,
    out_type=jax.ShapeDtypeStruct((NC, NA, FP), jnp.float32),
    scratch_types=[
        pltpu.VMEM((NWIN, W0), jnp.int32),
        pltpu.VMEM((NWIN, W0), jnp.int32),
        pltpu.VMEM((D_RING, W0, FP), jnp.float32),
        pltpu.VMEM_SHARED((NA, FP), jnp.float32),
        pltpu.SemaphoreType.DMA,
        pltpu.SemaphoreType.DMA,
    ],
    compiler_params=_CP,
)
def _scatter_kernel(src_hbm, dst_hbm, g_hbm, zeros_hbm, out_hbm,
                    sidx_v, didx_v, rows_v, acc_sh, sem_g, sem_s):
    c = lax.axis_index("c")
    s = lax.axis_index("s")
    wid = s * NC + c

    @pl.when((s == 0) & (c == 0))
    def _():
        pltpu.sync_copy(g_hbm, acc_sh)

    @pl.when((s == 0) & (c == 1))
    def _():
        pltpu.sync_copy(zeros_hbm, acc_sh)

    pltpu.sync_copy(src_hbm.at[wid], sidx_v)
    pltpu.sync_copy(dst_hbm.at[wid], didx_v)
    plsc.subcore_barrier()

    for i in range(P_PRE):
        pltpu.async_copy(g_hbm.at[sidx_v.at[i]], rows_v.at[i], sem_g)

    def w_body(w, carry):
        @pl.when(w >= P_PRE)
        def _():
            pltpu.make_async_copy(rows_v.at[0], acc_sh.at[didx_v.at[0]],
                                  sem_s).wait()

        pltpu.make_async_copy(g_hbm.at[sidx_v.at[w]],
                              rows_v.at[w % D_RING], sem_g).wait()
        pltpu.async_copy(rows_v.at[w % D_RING],
                         acc_sh.at[didx_v.at[w]], sem_s, add=True)
        nxt = w + P_PRE

        @pl.when(nxt < NWIN)
        def _():
            pltpu.async_copy(g_hbm.at[sidx_v.at[nxt]],
                             rows_v.at[nxt % D_RING], sem_g)

        return carry

    lax.fori_loop(0, NWIN, w_body, 0)
    for _ in range(P_PRE):
        pltpu.make_async_copy(rows_v.at[0], acc_sh.at[didx_v.at[0]],
                              sem_s).wait()
    plsc.subcore_barrier()

    @pl.when(s == 0)
    def _():
        pltpu.sync_copy(acc_sh, out_hbm.at[c])


def _prep_body(x_ref, w_ref, degp_ref, g_ref):
    deg = degp_ref[0, :N] + degp_ref[1, :N] + 1.0
    dis = lax.rsqrt(deg)[:, None]
    h = jnp.dot(x_ref[...], w_ref[...], preferred_element_type=jnp.float32)
    g_ref[pl.ds(0, N), :] = h * dis
    g_ref[pl.ds(N, NT), :] = jnp.zeros((NT, FP), jnp.float32)


def _fin_body(ap_ref, degp_ref, b_ref, lw_ref, lb_ref, o_ref):
    deg = degp_ref[0, :N] + degp_ref[1, :N] + 1.0
    dis = lax.rsqrt(deg)[:, None]
    a = (ap_ref[0, :N, :F_OUT] + ap_ref[1, :N, :F_OUT])
    out1 = a * dis + b_ref[...]
    nrm = jnp.sqrt(jnp.sum(out1 * out1, axis=1, keepdims=True))
    out1 = out1 / jnp.maximum(nrm, 1e-12)
    e = jnp.maximum(out1, 0.0)
    pmax = jnp.max(e, axis=0, keepdims=True)
    pmean = jnp.sum(e, axis=0, keepdims=True) * (1.0 / N)
    feat = jnp.concatenate([pmax, pmean], axis=1)
    o_ref[...] = jnp.dot(feat, lw_ref[...],
                         preferred_element_type=jnp.float32) + lb_ref[...]


_PAD = EP - E
_pad_src = ((np.arange(_PAD, dtype=np.int64) * 79) % N).astype(np.int32)
_pad_dst = (N + np.arange(_PAD, dtype=np.int64) % NT).astype(np.int32)


def kernel(x, edge_index, W, b, lin_W, lin_b):
    src = jnp.concatenate([edge_index[0], _pad_src]).reshape(NW, NWIN, W0)
    dst = jnp.concatenate([edge_index[1], _pad_dst]).reshape(NW, NWIN, W0)
    zeros_n = jnp.zeros((NA,), jnp.float32)
    zeros_nf = jnp.zeros((NA, FP), jnp.float32)
    w_pad = jnp.pad(W, ((0, 0), (0, FP - F_OUT)))

    degp = _deg_kernel(dst, zeros_n)

    g = pl.pallas_call(
        _prep_body,
        out_shape=jax.ShapeDtypeStruct((NA, FP), jnp.float32),
    )(x, w_pad, degp)

    ap = _scatter_kernel(src, dst, g, zeros_nf)

    out = pl.pallas_call(
        _fin_body,
        out_shape=jax.ShapeDtypeStruct((1, NUM_CLASSES), jnp.float32),
    )(ap, degp, b.reshape(1, F_OUT), lin_W, lin_b.reshape(1, NUM_CLASSES))
    return out


# speedup vs baseline: 80.0834x; 1.0317x over previous
import functools

import jax
import jax.numpy as jnp
import numpy as np
from jax import lax
from jax.experimental import pallas as pl
from jax.experimental.pallas import tpu as pltpu
from jax.experimental.pallas import tpu_sc as plsc

N = 10000
E = 320000
F_IN = 128
F_OUT = 20
FP = 24
NUM_CLASSES = 10

NC = 2
NS = 16
NW = NC * NS
W0 = 128
NWIN = 79
EP = NW * NWIN * W0
NT = 128
NA = N + NT

D_RING = 16
P_PRE = 8

_mesh = plsc.VectorSubcoreMesh(core_axis_name="c", subcore_axis_name="s")
_CP = pltpu.CompilerParams(use_tc_tiling_on_sc=False)


@functools.partial(
    pl.kernel,
    mesh=_mesh,
    out_type=jax.ShapeDtypeStruct((NC, NA), jnp.float32),
    scratch_types=[
        pltpu.VMEM((NWIN, W0), jnp.int32),
        pltpu.VMEM((W0,), jnp.float32),
        pltpu.VMEM_SHARED((NA,), jnp.float32),
        pltpu.SemaphoreType.DMA,
    ],
    compiler_params=_CP,
)
def _deg_kernel(dst_hbm, zeros_hbm, out_hbm, idx_v, ones_v, deg_sh, sem):
    c = lax.axis_index("c")
    s = lax.axis_index("s")
    wid = s * NC + c

    @pl.when(s == 0)
    def _():
        pltpu.sync_copy(zeros_hbm, deg_sh)

    for i in range(W0 // 16):
        ones_v[pl.ds(i * 16, 16)] = jnp.ones((16,), jnp.float32)
    pltpu.sync_copy(dst_hbm.at[wid], idx_v)
    plsc.subcore_barrier()

    def fire(w, carry):
        pltpu.async_copy(ones_v, deg_sh.at[idx_v.at[w]], sem, add=True)
        return carry

    lax.fori_loop(0, NWIN, fire, 0)

    def drain(w, carry):
        pltpu.make_async_copy(ones_v, deg_sh.at[idx_v.at[0]], sem).wait()
        return carry

    lax.fori_loop(0, NWIN, drain, 0)
    plsc.subcore_barrier()

    @pl.when(s == 0)
    def _():
        pltpu.sync_copy(deg_sh, out_hbm.at[c])


@functools.partial(
    pl.kernel,
    mesh=_mesh,
    out_type=jax.ShapeDtypeStruct((NC, NA, FP), jnp.float32),
    scratch_types=[
        pltpu.VMEM((NWIN, W0), jnp.int32),
        pltpu.VMEM((NWIN, W0), jnp.int32),
        pltpu.VMEM((D_RING, W0, FP), jnp.float32),
        pltpu.VMEM_SHARED((NA, FP), jnp.float32),
        pltpu.SemaphoreType.DMA,
        pltpu.SemaphoreType.DMA,
    ],
    compiler_params=_CP,
)
def _scatter_kernel(src_hbm, dst_hbm, g_hbm, zeros_hbm, out_hbm,
                    sidx_v, didx_v, rows_v, acc_sh, sem_g, sem_s):
    c = lax.axis_index("c")
    s = lax.axis_index("s")
    wid = s * NC + c

    @pl.when((s == 0) & (c == 0))
    def _():
        pltpu.sync_copy(g_hbm, acc_sh)

    @pl.when((s == 0) & (c == 1))
    def _():
        pltpu.sync_copy(zeros_hbm, acc_sh)

    pltpu.sync_copy(src_hbm.at[wid], sidx_v)
    pltpu.sync_copy(dst_hbm.at[wid], didx_v)
    plsc.subcore_barrier()

    for i in range(P_PRE):
        pltpu.async_copy(g_hbm.at[sidx_v.at[i]], rows_v.at[i], sem_g)

    def w_body(w, carry):
        @pl.when(w >= P_PRE)
        def _():
            pltpu.make_async_copy(rows_v.at[0], acc_sh.at[didx_v.at[0]],
                                  sem_s).wait()

        pltpu.make_async_copy(g_hbm.at[sidx_v.at[w]],
                              rows_v.at[w % D_RING], sem_g).wait()
        pltpu.async_copy(rows_v.at[w % D_RING],
                         acc_sh.at[didx_v.at[w]], sem_s, add=True)
        nxt = w + P_PRE

        @pl.when(nxt < NWIN)
        def _():
            pltpu.async_copy(g_hbm.at[sidx_v.at[nxt]],
                             rows_v.at[nxt % D_RING], sem_g)

        return carry

    lax.fori_loop(0, NWIN, w_body, 0)
    for _ in range(P_PRE):
        pltpu.make_async_copy(rows_v.at[0], acc_sh.at[didx_v.at[0]],
                              sem_s).wait()
    plsc.subcore_barrier()

    @pl.when(s == 0)
    def _():
        pltpu.sync_copy(acc_sh, out_hbm.at[c])


def _prep_body(x_ref, w_ref, degp_ref, g_ref):
    deg = degp_ref[0, :N] + degp_ref[1, :N] + 1.0
    dis = lax.rsqrt(deg)[:, None]
    h = jnp.dot(x_ref[...], w_ref[...], preferred_element_type=jnp.float32)
    g_ref[pl.ds(0, N), :] = h * dis
    g_ref[pl.ds(N, NT), :] = jnp.zeros((NT, FP), jnp.float32)


def _fin_body(ap_ref, degp_ref, b_ref, lw_ref, lb_ref, o_ref):
    deg = degp_ref[0, :N] + degp_ref[1, :N] + 1.0
    dis = lax.rsqrt(deg)[:, None]
    a = (ap_ref[0, :N, :F_OUT] + ap_ref[1, :N, :F_OUT])
    out1 = a * dis + b_ref[...]
    nrm = jnp.sqrt(jnp.sum(out1 * out1, axis=1, keepdims=True))
    out1 = out1 / jnp.maximum(nrm, 1e-12)
    e = jnp.maximum(out1, 0.0)
    pmax = jnp.max(e, axis=0, keepdims=True)
    pmean = jnp.sum(e, axis=0, keepdims=True) * (1.0 / N)
    feat = jnp.concatenate([pmax, pmean], axis=1)
    o_ref[...] = jnp.dot(feat, lw_ref[...],
                         preferred_element_type=jnp.float32) + lb_ref[...]


_PAD = EP - E
_pad_src = ((np.arange(_PAD, dtype=np.int64) * 79) % N).astype(np.int32)
_pad_dst = (N + np.arange(_PAD, dtype=np.int64) % NT).astype(np.int32)


def kernel(x, edge_index, W, b, lin_W, lin_b):
    src = jnp.concatenate([edge_index[0], _pad_src]).reshape(NW, NWIN, W0)
    dst = jnp.concatenate([edge_index[1], _pad_dst]).reshape(NW, NWIN, W0)
    zeros_n = jnp.zeros((NA,), jnp.float32)
    zeros_nf = jnp.zeros((NA, FP), jnp.float32)
    w_pad = jnp.pad(W, ((0, 0), (0, FP - F_OUT)))

    degp = _deg_kernel(dst, zeros_n)

    g = pl.pallas_call(
        _prep_body,
        out_shape=jax.ShapeDtypeStruct((NA, FP), jnp.float32),
    )(x, w_pad, degp)

    ap = _scatter_kernel(src, dst, g, zeros_nf)

    out = pl.pallas_call(
        _fin_body,
        out_shape=jax.ShapeDtypeStruct((1, NUM_CLASSES), jnp.float32),
    )(ap, degp, b.reshape(1, F_OUT), lin_W, lin_b.reshape(1, NUM_CLASSES))
    return out


# speedup vs baseline: 80.2415x; 1.0020x over previous
import functools

import jax
import jax.numpy as jnp
import numpy as np
from jax import lax
from jax.experimental import pallas as pl
from jax.experimental.pallas import tpu as pltpu
from jax.experimental.pallas import tpu_sc as plsc

N = 10000
E = 320000
F_IN = 128
F_OUT = 20
FP = 24
NUM_CLASSES = 10

NC = 2
NS = 16
NW = NC * NS
W0 = 128
NWIN = 79
EP = NW * NWIN * W0
NT = 128
NA = N + NT

D_RING = 24
P_PRE = 12

_mesh = plsc.VectorSubcoreMesh(core_axis_name="c", subcore_axis_name="s")
_CP = pltpu.CompilerParams(use_tc_tiling_on_sc=False)


@functools.partial(
    pl.kernel,
    mesh=_mesh,
    out_type=jax.ShapeDtypeStruct((NC, NA), jnp.float32),
    scratch_types=[
        pltpu.VMEM((NWIN, W0), jnp.int32),
        pltpu.VMEM((W0,), jnp.float32),
        pltpu.VMEM_SHARED((NA,), jnp.float32),
        pltpu.SemaphoreType.DMA,
    ],
    compiler_params=_CP,
)
def _deg_kernel(dst_hbm, zeros_hbm, out_hbm, idx_v, ones_v, deg_sh, sem):
    c = lax.axis_index("c")
    s = lax.axis_index("s")
    wid = s * NC + c

    @pl.when(s == 0)
    def _():
        pltpu.sync_copy(zeros_hbm, deg_sh)

    for i in range(W0 // 16):
        ones_v[pl.ds(i * 16, 16)] = jnp.ones((16,), jnp.float32)
    pltpu.sync_copy(dst_hbm.at[wid], idx_v)
    plsc.subcore_barrier()

    def fire(w, carry):
        pltpu.async_copy(ones_v, deg_sh.at[idx_v.at[w]], sem, add=True)
        return carry

    lax.fori_loop(0, NWIN, fire, 0)

    def drain(w, carry):
        pltpu.make_async_copy(ones_v, deg_sh.at[idx_v.at[0]], sem).wait()
        return carry

    lax.fori_loop(0, NWIN, drain, 0)
    plsc.subcore_barrier()

    @pl.when(s == 0)
    def _():
        pltpu.sync_copy(deg_sh, out_hbm.at[c])


@functools.partial(
    pl.kernel,
    mesh=_mesh,
    out_type=jax.ShapeDtypeStruct((NC, NA, FP), jnp.float32),
    scratch_types=[
        pltpu.VMEM((NWIN, W0), jnp.int32),
        pltpu.VMEM((NWIN, W0), jnp.int32),
        pltpu.VMEM((D_RING, W0, FP), jnp.float32),
        pltpu.VMEM_SHARED((NA, FP), jnp.float32),
        pltpu.SemaphoreType.DMA,
        pltpu.SemaphoreType.DMA,
    ],
    compiler_params=_CP,
)
def _scatter_kernel(src_hbm, dst_hbm, g_hbm, zeros_hbm, out_hbm,
                    sidx_v, didx_v, rows_v, acc_sh, sem_g, sem_s):
    c = lax.axis_index("c")
    s = lax.axis_index("s")
    wid = s * NC + c

    @pl.when((s == 0) & (c == 0))
    def _():
        pltpu.sync_copy(g_hbm, acc_sh)

    @pl.when((s == 0) & (c == 1))
    def _():
        pltpu.sync_copy(zeros_hbm, acc_sh)

    pltpu.sync_copy(src_hbm.at[wid], sidx_v)
    pltpu.sync_copy(dst_hbm.at[wid], didx_v)
    plsc.subcore_barrier()

    for i in range(P_PRE):
        pltpu.async_copy(g_hbm.at[sidx_v.at[i]], rows_v.at[i], sem_g)

    def w_body(w, carry):
        @pl.when(w >= P_PRE)
        def _():
            pltpu.make_async_copy(rows_v.at[0], acc_sh.at[didx_v.at[0]],
                                  sem_s).wait()

        pltpu.make_async_copy(g_hbm.at[sidx_v.at[w]],
                              rows_v.at[w % D_RING], sem_g).wait()
        pltpu.async_copy(rows_v.at[w % D_RING],
                         acc_sh.at[didx_v.at[w]], sem_s, add=True)
        nxt = w + P_PRE

        @pl.when(nxt < NWIN)
        def _():
            pltpu.async_copy(g_hbm.at[sidx_v.at[nxt]],
                             rows_v.at[nxt % D_RING], sem_g)

        return carry

    lax.fori_loop(0, NWIN, w_body, 0)
    for _ in range(P_PRE):
        pltpu.make_async_copy(rows_v.at[0], acc_sh.at[didx_v.at[0]],
                              sem_s).wait()
    plsc.subcore_barrier()

    @pl.when(s == 0)
    def _():
        pltpu.sync_copy(acc_sh, out_hbm.at[c])


def _prep_body(x_ref, w_ref, degp_ref, g_ref):
    deg = degp_ref[0, :N] + degp_ref[1, :N] + 1.0
    dis = lax.rsqrt(deg)[:, None]
    h = jnp.dot(x_ref[...], w_ref[...], preferred_element_type=jnp.float32)
    g_ref[pl.ds(0, N), :] = h * dis
    g_ref[pl.ds(N, NT), :] = jnp.zeros((NT, FP), jnp.float32)


def _fin_body(ap_ref, degp_ref, b_ref, lw_ref, lb_ref, o_ref):
    deg = degp_ref[0, :N] + degp_ref[1, :N] + 1.0
    dis = lax.rsqrt(deg)[:, None]
    a = (ap_ref[0, :N, :F_OUT] + ap_ref[1, :N, :F_OUT])
    out1 = a * dis + b_ref[...]
    nrm = jnp.sqrt(jnp.sum(out1 * out1, axis=1, keepdims=True))
    out1 = out1 / jnp.maximum(nrm, 1e-12)
    e = jnp.maximum(out1, 0.0)
    pmax = jnp.max(e, axis=0, keepdims=True)
    pmean = jnp.sum(e, axis=0, keepdims=True) * (1.0 / N)
    feat = jnp.concatenate([pmax, pmean], axis=1)
    o_ref[...] = jnp.dot(feat, lw_ref[...],
                         preferred_element_type=jnp.float32) + lb_ref[...]


_PAD = EP - E
_pad_src = ((np.arange(_PAD, dtype=np.int64) * 79) % N).astype(np.int32)
_pad_dst = (N + np.arange(_PAD, dtype=np.int64) % NT).astype(np.int32)


def kernel(x, edge_index, W, b, lin_W, lin_b):
    src = jnp.concatenate([edge_index[0], _pad_src]).reshape(NW, NWIN, W0)
    dst = jnp.concatenate([edge_index[1], _pad_dst]).reshape(NW, NWIN, W0)
    zeros_n = jnp.zeros((NA,), jnp.float32)
    zeros_nf = jnp.zeros((NA, FP), jnp.float32)
    w_pad = jnp.pad(W, ((0, 0), (0, FP - F_OUT)))

    degp = _deg_kernel(dst, zeros_n)

    g = pl.pallas_call(
        _prep_body,
        out_shape=jax.ShapeDtypeStruct((NA, FP), jnp.float32),
    )(x, w_pad, degp)

    ap = _scatter_kernel(src, dst, g, zeros_nf)

    out = pl.pallas_call(
        _fin_body,
        out_shape=jax.ShapeDtypeStruct((1, NUM_CLASSES), jnp.float32),
    )(ap, degp, b.reshape(1, F_OUT), lin_W, lin_b.reshape(1, NUM_CLASSES))
    return out
